# baseline (speedup 1.0000x reference)
import jax
import jax.numpy as jnp
from jax import lax
from jax.experimental import pallas as pl
from jax.experimental.pallas import tpu as pltpu

_B, _T, _C = 4, 512, 768
_G, _V = 2, 320
_TE = 64
_GV = _G * _V
_ROWS = _B * _T
_CHUNKS = _B * _G
_RPC = _ROWS // _CHUNKS
_FLAT = _RPC * _GV
_EMB = _G * _TE


_CPS = 2
_STEPS = _CHUNKS // _CPS


def _vq_body(x_ref, w_ref, b_ref, cb_ref, out_ref):
    w = w_ref[...]
    bvec = b_ref[...]
    rows = lax.broadcasted_iota(jnp.int32, (_RPC, _GV), 0)
    cols = lax.broadcasted_iota(jnp.int32, (_RPC, _GV), 1)
    flat = rows * _GV + cols
    for h in range(_CPS):
        x = x_ref[pl.ds(h * _RPC, _RPC), :]
        p = lax.dot_general(x, w, (((1,), (1,)), ((), ())),
                            preferred_element_type=jnp.float32)
        p = p + bvec
        m = jnp.max(p)
        k = jnp.min(jnp.where(p == m, flat, _FLAT))
        r = k // _GV
        c = k - r * _GV
        g = c // _V
        row64 = cb_ref[pl.ds(c, 1), :]
        zero64 = jnp.zeros((1, _TE), jnp.float32)
        rowfull = jnp.concatenate(
            [jnp.where(g == 0, row64, zero64),
             jnp.where(g == 1, row64, zero64)], axis=1)
        out_ref[pl.ds(h * _RPC, _RPC), :] = jnp.zeros((_RPC, _EMB), jnp.float32)
        out_ref[pl.ds(h * _RPC + r, 1), :] = rowfull


def kernel(X, W, b, codebook):
    X2 = X.reshape(_ROWS, _C)
    cb = codebook.reshape(_GV, _TE)
    b2 = b.reshape(1, _GV)
    out = pl.pallas_call(
        _vq_body,
        grid=(_STEPS,),
        in_specs=[
            pl.BlockSpec((_CPS * _RPC, _C), lambda j: (j, 0)),
            pl.BlockSpec((_GV, _C), lambda j: (0, 0)),
            pl.BlockSpec((1, _GV), lambda j: (0, 0)),
            pl.BlockSpec((_GV, _TE), lambda j: (0, 0)),
        ],
        out_specs=pl.BlockSpec((_CPS * _RPC, _EMB), lambda j: (j, 0)),
        out_shape=jax.ShapeDtypeStruct((_ROWS, _EMB), jnp.float32),
        compiler_params=pltpu.CompilerParams(
            dimension_semantics=("arbitrary",)),
    )(X2, W, b2, cb)
    return out.reshape(_B, _T, _EMB)


# speedup vs baseline: 1.0998x; 1.0998x over previous
import jax
import jax.numpy as jnp
from jax import lax
from jax.experimental import pallas as pl
from jax.experimental.pallas import tpu as pltpu

_B, _T, _C = 4, 512, 768
_G, _V = 2, 320
_TE = 64
_GV = _G * _V
_ROWS = _B * _T
_CHUNKS = _B * _G
_RPC = _ROWS // _CHUNKS
_FLAT = _RPC * _GV
_EMB = _G * _TE


_CPS = 4
_STEPS = _CHUNKS // _CPS


def _vq_body(x_ref, w_ref, b_ref, cb_ref, out_ref):
    w = w_ref[...]
    bvec = b_ref[...]
    rows = lax.broadcasted_iota(jnp.int32, (_RPC, _GV), 0)
    cols = lax.broadcasted_iota(jnp.int32, (_RPC, _GV), 1)
    flat = rows * _GV + cols
    for h in range(_CPS):
        x = x_ref[pl.ds(h * _RPC, _RPC), :]
        p = lax.dot_general(x, w, (((1,), (1,)), ((), ())),
                            preferred_element_type=jnp.float32)
        p = p + bvec
        m = jnp.max(p)
        k = jnp.min(jnp.where(p == m, flat, _FLAT))
        r = k // _GV
        c = k - r * _GV
        g = c // _V
        row64 = cb_ref[pl.ds(c, 1), :]
        zero64 = jnp.zeros((1, _TE), jnp.float32)
        rowfull = jnp.concatenate(
            [jnp.where(g == 0, row64, zero64),
             jnp.where(g == 1, row64, zero64)], axis=1)
        out_ref[pl.ds(h * _RPC, _RPC), :] = jnp.zeros((_RPC, _EMB), jnp.float32)
        out_ref[pl.ds(h * _RPC + r, 1), :] = rowfull


def kernel(X, W, b, codebook):
    X2 = X.reshape(_ROWS, _C)
    cb = codebook.reshape(_GV, _TE)
    b2 = b.reshape(1, _GV)
    out = pl.pallas_call(
        _vq_body,
        grid=(_STEPS,),
        in_specs=[
            pl.BlockSpec((_CPS * _RPC, _C), lambda j: (j, 0)),
            pl.BlockSpec((_GV, _C), lambda j: (0, 0)),
            pl.BlockSpec((1, _GV), lambda j: (0, 0)),
            pl.BlockSpec((_GV, _TE), lambda j: (0, 0)),
        ],
        out_specs=pl.BlockSpec((_CPS * _RPC, _EMB), lambda j: (j, 0)),
        out_shape=jax.ShapeDtypeStruct((_ROWS, _EMB), jnp.float32),
        compiler_params=pltpu.CompilerParams(
            dimension_semantics=("arbitrary",)),
    )(X2, W, b2, cb)
    return out.reshape(_B, _T, _EMB)
